# baseline (speedup 1.0000x reference)
import functools

import jax
import jax.numpy as jnp
from jax import lax
from jax.experimental import pallas as pl
from jax.experimental.pallas import tpu as pltpu
from jax.experimental.pallas import tpu_sc as plsc

BATCH = 4096
SEQ = 200
EMB = 64
NCLS = 8
VOCAB = 1000000

NC = 2
NS = 16
NW = NC * NS
ROWS_PER_W = BATCH // NW
CPR = 5
CHUNK = SEQ // CPR
CHUNKS_PER_W = ROWS_PER_W * CPR
L = 16
EV = EMB // L

NBUF = 20

_mesh = plsc.VectorSubcoreMesh(
    core_axis_name="c", subcore_axis_name="s", num_cores=NC, num_subcores=NS)


@functools.partial(
    pl.kernel,
    out_type=jax.ShapeDtypeStruct((BATCH, EMB), jnp.float32),
    mesh=_mesh,
    scratch_types=[
        pltpu.VMEM((CHUNKS_PER_W, CHUNK), jnp.int32),
        pltpu.VMEM((NBUF, CHUNK, EMB), jnp.float32),
        pltpu.VMEM((ROWS_PER_W, EMB), jnp.float32),
        [pltpu.SemaphoreType.DMA] * NBUF,
    ],
    compiler_params=pltpu.CompilerParams(use_tc_tiling_on_sc=False),
)
def _pool_kernel(idx_hbm, table_hbm, out_hbm, idx_v, buf, out_v, sems):
    wid = lax.axis_index("s") * NC + lax.axis_index("c")
    base = wid * CHUNKS_PER_W
    pltpu.sync_copy(idx_hbm.at[pl.ds(base, CHUNKS_PER_W)], idx_v)

    neg = jnp.full((L,), -jnp.inf, dtype=jnp.float32)

    def fire(chunk, slot):
        pltpu.async_copy(table_hbm.at[idx_v.at[chunk]], buf.at[slot],
                         sems[slot])

    def drain(slot):
        pltpu.make_async_copy(table_hbm.at[pl.ds(0, CHUNK)],
                              buf.at[slot], sems[slot]).wait()

    def chunk_max(bc, a):
        @plsc.parallel_loop(0, CHUNK // 2, unroll=4, carry=a)
        def acc(j, a):
            lo = tuple(
                jnp.maximum(a[d], bc[2 * j, pl.ds(L * d, L)])
                for d in range(EV))
            hi = tuple(
                jnp.maximum(a[EV + d], bc[2 * j + 1, pl.ds(L * d, L)])
                for d in range(EV))
            return lo + hi
        return acc

    for slot in range(NBUF):
        fire(slot, slot)

    def outer_body(k, carry):
        acc = carry
        for p in range(NBUF):
            c = NBUF * k + p
            drain(p)
            if p % CPR == 0:
                acc = (neg,) * (2 * EV)
            acc = chunk_max(buf.at[p], acc)
            if p % CPR == CPR - 1:
                r = (c - (CPR - 1)) // CPR
                for d in range(EV):
                    out_v[r, pl.ds(L * d, L)] = jnp.maximum(
                        acc[d], acc[EV + d])
            fire(lax.rem(c + NBUF, CHUNKS_PER_W), p)
        return acc

    lax.fori_loop(0, CHUNKS_PER_W // NBUF, outer_body, (neg,) * (2 * EV))
    for slot in range(NBUF):
        drain(slot)
    pltpu.sync_copy(out_v, out_hbm.at[pl.ds(wid * ROWS_PER_W, ROWS_PER_W)])


def _matmul_body(x_ref, wt_ref, b_ref, o_ref):
    o_ref[:] = (
        jnp.dot(x_ref[:], wt_ref[:], preferred_element_type=jnp.float32)
        + b_ref[:])


_matmul = pl.pallas_call(
    _matmul_body,
    out_shape=jax.ShapeDtypeStruct((BATCH, NCLS), jnp.float32),
)


def kernel(content, emb_table, W, b):
    idx = content.reshape(BATCH * CPR, CHUNK)
    pooled = _pool_kernel(idx, emb_table)
    return _matmul(pooled, W.T, b.reshape(1, NCLS))
